# baseline (speedup 1.0000x reference)
import jax
import jax.numpy as jnp
from jax import lax
from jax.experimental import pallas as pl
from jax.experimental.pallas import tpu as pltpu
from jax.experimental.pallas import tpu_sc as plsc

N_MET = 100000
N_RXN = 100000
E_SUB = 800000
E_ALL = 1600000

NC, NS, NW = 2, 16, 32
ES_PAD = 819200
ES_ROWS = ES_PAD // 128
EA_PAD = 1638400
EA_ROWS = EA_PAD // 128
NR_PAD = 100352
NM_PAD = 100352
RPT = NR_PAD // NS

_MESH = plsc.VectorSubcoreMesh(core_axis_name="c", subcore_axis_name="s")
_f32 = jnp.float32


def _zero_fill(buf, n16):
    def body(i, _):
        buf[pl.ds(i * 16, 16)] = jnp.zeros((16,), _f32)
        return 0
    lax.fori_loop(0, n16, body, 0)


def _gather_scatter_sub(conc_hbm, ext_hbm, met2d, rxn2d,
                        concg, extp, cntp,
                        idx_m, idx_r, vals_c, vals_e, ones_v, zbuf,
                        conc_sh, ext_sh, ext_acc, cnt_acc, sem):
    c = lax.axis_index("c")
    s = lax.axis_index("s")
    wid = c * NS + s

    def fill_ones(i, _):
        ones_v[i // 8, pl.ds((i % 8) * 16, 16)] = jnp.full((16,), 1.0, _f32)
        return 0
    lax.fori_loop(0, 320, fill_ones, 0)
    _zero_fill(zbuf, RPT // 16)
    pltpu.sync_copy(zbuf, ext_acc.at[pl.ds(s * RPT, RPT)])
    pltpu.sync_copy(zbuf, cnt_acc.at[pl.ds(s * RPT, RPT)])
    pltpu.sync_copy(conc_hbm.at[pl.ds(s * RPT, RPT)], conc_sh.at[pl.ds(s * RPT, RPT)])
    pltpu.sync_copy(ext_hbm.at[pl.ds(s * RPT, RPT)], ext_sh.at[pl.ds(s * RPT, RPT)])
    plsc.subcore_barrier()

    base = jnp.where(c == 1, s * 120, 1920 + s * 280)
    nk = jnp.where(c == 1, 3, 7)

    def chunk(k, _):
        off = base + k * 40
        pltpu.sync_copy(met2d.at[pl.ds(off, 40)], idx_m)
        pltpu.sync_copy(rxn2d.at[pl.ds(off, 40)], idx_r)

        def grow(j, _):
            pltpu.async_copy(conc_sh.at[idx_m.at[j]], vals_c.at[j], sem)
            pltpu.async_copy(ext_sh.at[idx_m.at[j]], vals_e.at[j], sem)
            return 0
        lax.fori_loop(0, 40, grow, 0)
        pltpu.make_async_copy(met2d.at[pl.ds(0, 40)], idx_m, sem).wait()
        pltpu.make_async_copy(met2d.at[pl.ds(0, 40)], idx_m, sem).wait()
        pltpu.sync_copy(vals_c, concg.at[pl.ds(off, 40)])

        def srow(j, _):
            pltpu.async_copy(vals_e.at[j], ext_acc.at[idx_r.at[j]], sem, add=True)
            pltpu.async_copy(ones_v.at[j], cnt_acc.at[idx_r.at[j]], sem, add=True)
            return 0
        lax.fori_loop(0, 40, srow, 0)
        pltpu.make_async_copy(met2d.at[pl.ds(0, 40)], idx_m, sem).wait()
        pltpu.make_async_copy(met2d.at[pl.ds(0, 40)], idx_m, sem).wait()
        return 0
    lax.fori_loop(0, nk, chunk, 0)
    plsc.subcore_barrier()
    dst = c * NR_PAD + s * RPT
    pltpu.sync_copy(ext_acc.at[pl.ds(s * RPT, RPT)], extp.at[pl.ds(dst, RPT)])
    pltpu.sync_copy(cnt_acc.at[pl.ds(s * RPT, RPT)], cntp.at[pl.ds(dst, RPT)])


def _scatter_msg(msgt_hbm, rxn2d, hp,
                 idx_r, msg_tv, zbuf,
                 a0, a1, a2, a3, a4, a5, a6, a7,
                 a8, a9, a10, a11, a12, a13, a14, a15, sem):
    c = lax.axis_index("c")
    s = lax.axis_index("s")
    accs = [a0, a1, a2, a3, a4, a5, a6, a7,
            a8, a9, a10, a11, a12, a13, a14, a15]

    _zero_fill(zbuf, RPT // 16)
    for k in range(16):
        pltpu.sync_copy(zbuf, accs[k].at[pl.ds(s * RPT, RPT)])
    plsc.subcore_barrier()

    base = jnp.where(c == 1, s * 120, 1920 + s * 280)
    nk = jnp.where(c == 1, 15, 35)

    def chunk(kk, _):
        off = base + kk * 8
        pltpu.sync_copy(rxn2d.at[pl.ds(off, 8)], idx_r)
        pltpu.sync_copy(msgt_hbm.at[:, pl.ds(off * 128, 1024)], msg_tv)

        def srow(j, _):
            for k in range(16):
                pltpu.async_copy(msg_tv.at[k, pl.ds(j * 128, 128)],
                                 accs[k].at[idx_r.at[j]], sem, add=True)
            return 0
        lax.fori_loop(0, 8, srow, 0)
        pltpu.make_async_copy(msgt_hbm.at[:, pl.ds(0, 1024)], msg_tv, sem).wait()
        return 0
    lax.fori_loop(0, nk, chunk, 0)
    plsc.subcore_barrier()
    for k in range(16):
        pltpu.sync_copy(accs[k].at[pl.ds(s * RPT, RPT)],
                        hp.at[k, pl.ds(c * NR_PAD + s * RPT, RPT)])


def _scatter_flux(v_hbm, rxn2d, met2d, sto2d, dxp,
                  idx_r, idx_m, sto_v, val_v, zbuf, v_sh, dx_acc, sem):
    c = lax.axis_index("c")
    s = lax.axis_index("s")
    wid = c * NS + s

    _zero_fill(zbuf, RPT // 16)
    pltpu.sync_copy(zbuf, dx_acc.at[pl.ds(s * RPT, RPT)])
    pltpu.sync_copy(v_hbm.at[pl.ds(s * RPT, RPT)], v_sh.at[pl.ds(s * RPT, RPT)])
    plsc.subcore_barrier()

    base = jnp.where(c == 1, s * 240, 3840 + s * 560)
    nk = jnp.where(c == 1, 6, 14)

    def chunk(k, _):
        off = base + k * 40
        pltpu.sync_copy(rxn2d.at[pl.ds(off, 40)], idx_r)
        pltpu.sync_copy(met2d.at[pl.ds(off, 40)], idx_m)
        pltpu.sync_copy(sto2d.at[pl.ds(off, 40)], sto_v)

        def grow(j, _):
            pltpu.async_copy(v_sh.at[idx_r.at[j]], val_v.at[j], sem)
            return 0
        lax.fori_loop(0, 40, grow, 0)
        pltpu.make_async_copy(sto2d.at[pl.ds(0, 40)], val_v, sem).wait()

        def fmul(i, _):
            r = i // 8
            l = (i % 8) * 16
            val_v[r, pl.ds(l, 16)] = val_v[r, pl.ds(l, 16)] * sto_v[r, pl.ds(l, 16)]
            return 0
        lax.fori_loop(0, 320, fmul, 0)

        def srow(j, _):
            pltpu.async_copy(val_v.at[j], dx_acc.at[idx_m.at[j]], sem, add=True)
            return 0
        lax.fori_loop(0, 40, srow, 0)
        pltpu.make_async_copy(sto2d.at[pl.ds(0, 40)], val_v, sem).wait()
        return 0
    lax.fori_loop(0, nk, chunk, 0)
    plsc.subcore_barrier()
    pltpu.sync_copy(dx_acc.at[pl.ds(s * RPT, RPT)],
                    dxp.at[pl.ds(c * NM_PAD + s * RPT, RPT)])


def _msg_mlp_block(c_ref, s_ref, wc_ref, ws_ref, b1c_ref, W2_ref, out_ref):
    cr = c_ref[...]
    sr = s_ref[...]
    ones = jnp.ones(cr.shape, _f32)
    z = (jax.lax.dot_general(wc_ref[...], cr, (((1,), (0,)), ((), ())),
                             preferred_element_type=_f32)
         + jax.lax.dot_general(ws_ref[...], sr, (((1,), (0,)), ((), ())),
                               preferred_element_type=_f32)
         + jax.lax.dot_general(b1c_ref[...], ones, (((1,), (0,)), ((), ())),
                               preferred_element_type=_f32))
    h = jnp.tanh(z)
    out_ref[...] = jax.lax.dot_general(W2_ref[...], h, (((1,), (0,)), ((), ())),
                                       preferred_element_type=_f32)


def _rate_block(h0_ref, h1_ref, ee_ref, nn_ref, lk_ref,
                b2c_ref, V1a_ref, V2_ref, c2_ref, v_ref):
    ee = ee_ref[...]
    nn = nn_ref[...]
    n = nn[0:1, :] + nn[1:2, :]
    h = (h0_ref[...] + h1_ref[...]
         + jax.lax.dot_general(b2c_ref[...], n, (((1,), (0,)), ((), ())),
                               preferred_element_type=_f32))
    ones = jnp.ones(n.shape, _f32)
    h_aug = jnp.concatenate([h, ones], axis=0)
    t = jnp.tanh(jax.lax.dot_general(V1a_ref[...], h_aug, (((1,), (0,)), ((), ())),
                                     preferred_element_type=_f32))
    r = jax.lax.dot_general(V2_ref[...], t, (((1,), (0,)), ((), ())),
                            preferred_element_type=_f32) + c2_ref[0, 0]
    base_v = jax.nn.softplus(r)
    ext_mean = (ee[0:1, :] + ee[1:2, :]) / jnp.maximum(n, 1.0)
    k = jnp.exp(lk_ref[...] * jnp.log(10.0).astype(_f32))
    v_ref[...] = k * ext_mean * base_v


def _add_block(p_ref, o_ref):
    p = p_ref[...]
    o_ref[...] = p[0:1, :] + p[1:2, :]


def kernel(x, met_sub, rxn_sub, sto_sub, met_all, rxn_all, sto_all,
           W1, b1, W2, b2, V1, c1, V2, c2, log_k):
    conc = jnp.concatenate([x[:, 3], jnp.zeros((NR_PAD - N_MET,), _f32)])
    ext = jnp.concatenate([x[:, 4], jnp.zeros((NR_PAD - N_MET,), _f32)])

    ps = ES_PAD - E_SUB
    met2d_s = jnp.concatenate([met_sub, jnp.zeros((ps,), jnp.int32)]).reshape(ES_ROWS, 128)
    rxn2d_s = jnp.concatenate([rxn_sub, jnp.full((ps,), N_RXN, jnp.int32)]).reshape(ES_ROWS, 128)
    sto_s = jnp.concatenate([sto_sub, jnp.zeros((ps,), _f32)])
    pa = EA_PAD - E_ALL
    rxn2d_a = jnp.concatenate([rxn_all, jnp.zeros((pa,), jnp.int32)]).reshape(EA_ROWS, 128)
    met2d_a = jnp.concatenate([met_all, jnp.full((pa,), N_MET, jnp.int32)]).reshape(EA_ROWS, 128)
    sto2d_a = jnp.concatenate([sto_all, jnp.zeros((pa,), _f32)]).reshape(EA_ROWS, 128)
    lk_pad = jnp.concatenate([log_k, jnp.zeros((NR_PAD - N_RXN,), _f32)])

    kernel_a = pl.kernel(
        _gather_scatter_sub,
        out_type=(jax.ShapeDtypeStruct((ES_ROWS, 128), _f32),
                  jax.ShapeDtypeStruct((NC * NR_PAD,), _f32),
                  jax.ShapeDtypeStruct((NC * NR_PAD,), _f32)),
        mesh=_MESH,
        scratch_types=[
            pltpu.VMEM((40, 128), jnp.int32),
            pltpu.VMEM((40, 128), jnp.int32),
            pltpu.VMEM((40, 128), _f32),
            pltpu.VMEM((40, 128), _f32),
            pltpu.VMEM((40, 128), _f32),
            pltpu.VMEM((RPT,), _f32),
            pltpu.VMEM_SHARED((NR_PAD,), _f32),
            pltpu.VMEM_SHARED((NR_PAD,), _f32),
            pltpu.VMEM_SHARED((NR_PAD,), _f32),
            pltpu.VMEM_SHARED((NR_PAD,), _f32),
            pltpu.SemaphoreType.DMA,
        ],
    )
    concg2d, extp, cntp = kernel_a(conc, ext, met2d_s, rxn2d_s)
    concg = concg2d.reshape(ES_PAD)

    BE = 8192
    msg_t = pl.pallas_call(
        _msg_mlp_block,
        grid=(ES_PAD // BE,),
        in_specs=[
            pl.BlockSpec((1, BE), lambda i: (0, i)),
            pl.BlockSpec((1, BE), lambda i: (0, i)),
            pl.BlockSpec((32, 1), lambda i: (0, 0)),
            pl.BlockSpec((32, 1), lambda i: (0, 0)),
            pl.BlockSpec((32, 1), lambda i: (0, 0)),
            pl.BlockSpec((16, 32), lambda i: (0, 0)),
        ],
        out_specs=pl.BlockSpec((16, BE), lambda i: (0, i)),
        out_shape=jax.ShapeDtypeStruct((16, ES_PAD), _f32),
    )(concg[None, :], sto_s[None, :], W1[:, 0:1], W1[:, 1:2], b1[:, None], W2)

    kernel_c = pl.kernel(
        _scatter_msg,
        out_type=jax.ShapeDtypeStruct((16, NC * NR_PAD), _f32),
        mesh=_MESH,
        scratch_types=(
            [pltpu.VMEM((8, 128), jnp.int32),
             pltpu.VMEM((16, 1024), _f32),
             pltpu.VMEM((RPT,), _f32)]
            + [pltpu.VMEM_SHARED((NR_PAD,), _f32) for _ in range(16)]
            + [pltpu.SemaphoreType.DMA]
        ),
    )
    hp = kernel_c(msg_t, rxn2d_s)

    V1aug = jnp.concatenate([V1, c1[:, None]], axis=1)
    extp2 = extp.reshape(NC, NR_PAD)
    cntp2 = cntp.reshape(NC, NR_PAD)
    BR = 2048
    v_t = pl.pallas_call(
        _rate_block,
        grid=(NR_PAD // BR,),
        in_specs=[
            pl.BlockSpec((16, BR), lambda i: (0, i)),
            pl.BlockSpec((16, BR), lambda i: (0, i)),
            pl.BlockSpec((2, BR), lambda i: (0, i)),
            pl.BlockSpec((2, BR), lambda i: (0, i)),
            pl.BlockSpec((1, BR), lambda i: (0, i)),
            pl.BlockSpec((16, 1), lambda i: (0, 0)),
            pl.BlockSpec((32, 17), lambda i: (0, 0)),
            pl.BlockSpec((1, 32), lambda i: (0, 0)),
            pl.BlockSpec((1, 1), lambda i: (0, 0)),
        ],
        out_specs=pl.BlockSpec((1, BR), lambda i: (0, i)),
        out_shape=jax.ShapeDtypeStruct((1, NR_PAD), _f32),
    )(hp[:, :NR_PAD], hp[:, NR_PAD:], extp2, cntp2,
      lk_pad[None, :], b2[:, None], V1aug, V2, c2[:, None])

    kernel_e = pl.kernel(
        _scatter_flux,
        out_type=jax.ShapeDtypeStruct((NC * NM_PAD,), _f32),
        mesh=_MESH,
        scratch_types=[
            pltpu.VMEM((40, 128), jnp.int32),
            pltpu.VMEM((40, 128), jnp.int32),
            pltpu.VMEM((40, 128), _f32),
            pltpu.VMEM((40, 128), _f32),
            pltpu.VMEM((RPT,), _f32),
            pltpu.VMEM_SHARED((NR_PAD,), _f32),
            pltpu.VMEM_SHARED((NM_PAD,), _f32),
            pltpu.SemaphoreType.DMA,
        ],
    )
    dxp = kernel_e(v_t.reshape(NR_PAD), rxn2d_a, met2d_a, sto2d_a)

    BN = 2048
    dx = pl.pallas_call(
        _add_block,
        grid=(NM_PAD // BN,),
        in_specs=[pl.BlockSpec((2, BN), lambda i: (0, i))],
        out_specs=pl.BlockSpec((1, BN), lambda i: (0, i)),
        out_shape=jax.ShapeDtypeStruct((1, NM_PAD), _f32),
    )(dxp.reshape(NC, NM_PAD))

    return dx.reshape(NM_PAD)[:N_MET, None]


# speedup vs baseline: 1.1557x; 1.1557x over previous
import jax
import jax.numpy as jnp
from jax import lax
from jax.experimental import pallas as pl
from jax.experimental.pallas import tpu as pltpu
from jax.experimental.pallas import tpu_sc as plsc

N_MET = 100000
N_RXN = 100000
E_SUB = 800000
E_ALL = 1600000

NC, NS, NW = 2, 16, 32
ES_PAD = 819200
ES_ROWS = ES_PAD // 128
EA_PAD = 1638400
EA_ROWS = EA_PAD // 128
NR_PAD = 100352
NM_PAD = 100352
RPT = NR_PAD // NS

_MESH = plsc.VectorSubcoreMesh(core_axis_name="c", subcore_axis_name="s")
_f32 = jnp.float32


def _zero_fill(buf, n16):
    def body(i, _):
        buf[pl.ds(i * 16, 16)] = jnp.zeros((16,), _f32)
        return 0
    lax.fori_loop(0, n16, body, 0)


def _gather_scatter_sub(conc_hbm, ext_hbm, met2d, rxn2d,
                        concg, extp, cntp,
                        idx_m, idx_r, vals_c, vals_e, ones_v, zbuf,
                        conc_sh, ext_sh, ext_acc, cnt_acc, sem):
    c = lax.axis_index("c")
    s = lax.axis_index("s")
    wid = c * NS + s

    def fill_ones(i, _):
        ones_v[i // 8, pl.ds((i % 8) * 16, 16)] = jnp.full((16,), 1.0, _f32)
        return 0
    lax.fori_loop(0, 320, fill_ones, 0)
    _zero_fill(zbuf, RPT // 16)
    pltpu.sync_copy(zbuf, ext_acc.at[pl.ds(s * RPT, RPT)])
    pltpu.sync_copy(zbuf, cnt_acc.at[pl.ds(s * RPT, RPT)])
    pltpu.sync_copy(conc_hbm.at[pl.ds(s * RPT, RPT)], conc_sh.at[pl.ds(s * RPT, RPT)])
    pltpu.sync_copy(ext_hbm.at[pl.ds(s * RPT, RPT)], ext_sh.at[pl.ds(s * RPT, RPT)])
    plsc.subcore_barrier()

    base = wid * 200
    nk = 5

    def chunk(k, _):
        off = base + k * 40
        pltpu.sync_copy(met2d.at[pl.ds(off, 40)], idx_m)
        pltpu.sync_copy(rxn2d.at[pl.ds(off, 40)], idx_r)

        def grow(j, _):
            pltpu.async_copy(conc_sh.at[idx_m.at[j]], vals_c.at[j], sem)
            pltpu.async_copy(ext_sh.at[idx_m.at[j]], vals_e.at[j], sem)
            return 0
        lax.fori_loop(0, 40, grow, 0)
        pltpu.make_async_copy(met2d.at[pl.ds(0, 40)], idx_m, sem).wait()
        pltpu.make_async_copy(met2d.at[pl.ds(0, 40)], idx_m, sem).wait()
        pltpu.sync_copy(vals_c, concg.at[pl.ds(off, 40)])

        def srow(j, _):
            pltpu.async_copy(vals_e.at[j], ext_acc.at[idx_r.at[j]], sem, add=True)
            pltpu.async_copy(ones_v.at[j], cnt_acc.at[idx_r.at[j]], sem, add=True)
            return 0
        lax.fori_loop(0, 40, srow, 0)
        pltpu.make_async_copy(met2d.at[pl.ds(0, 40)], idx_m, sem).wait()
        pltpu.make_async_copy(met2d.at[pl.ds(0, 40)], idx_m, sem).wait()
        return 0
    lax.fori_loop(0, nk, chunk, 0)
    plsc.subcore_barrier()
    dst = c * NR_PAD + s * RPT
    pltpu.sync_copy(ext_acc.at[pl.ds(s * RPT, RPT)], extp.at[pl.ds(dst, RPT)])
    pltpu.sync_copy(cnt_acc.at[pl.ds(s * RPT, RPT)], cntp.at[pl.ds(dst, RPT)])


def _scatter_msg(msgt_hbm, rxn2d, hp,
                 idx_r, msg_tv, zbuf,
                 a0, a1, a2, a3, a4, a5, a6, a7,
                 a8, a9, a10, a11, a12, a13, a14, a15, sem):
    c = lax.axis_index("c")
    s = lax.axis_index("s")
    wid = c * NS + s
    accs = [a0, a1, a2, a3, a4, a5, a6, a7,
            a8, a9, a10, a11, a12, a13, a14, a15]

    _zero_fill(zbuf, RPT // 16)
    for k in range(16):
        pltpu.sync_copy(zbuf, accs[k].at[pl.ds(s * RPT, RPT)])
    plsc.subcore_barrier()

    base = wid * 200
    nk = 25

    def chunk(kk, _):
        off = base + kk * 8
        pltpu.sync_copy(rxn2d.at[pl.ds(off, 8)], idx_r)
        pltpu.sync_copy(msgt_hbm.at[:, pl.ds(off * 128, 1024)], msg_tv)

        def srow(j, _):
            for k in range(16):
                pltpu.async_copy(msg_tv.at[k, pl.ds(j * 128, 128)],
                                 accs[k].at[idx_r.at[j]], sem, add=True)
            return 0
        lax.fori_loop(0, 8, srow, 0)
        pltpu.make_async_copy(msgt_hbm.at[:, pl.ds(0, 1024)], msg_tv, sem).wait()
        return 0
    lax.fori_loop(0, nk, chunk, 0)
    plsc.subcore_barrier()
    for k in range(16):
        pltpu.sync_copy(accs[k].at[pl.ds(s * RPT, RPT)],
                        hp.at[k, pl.ds(c * NR_PAD + s * RPT, RPT)])


def _scatter_flux(v_hbm, rxn2d, met2d, sto2d, dxp,
                  idx_r, idx_m, sto_v, val_v, zbuf, v_sh, dx_acc, sem):
    c = lax.axis_index("c")
    s = lax.axis_index("s")
    wid = c * NS + s

    _zero_fill(zbuf, RPT // 16)
    pltpu.sync_copy(zbuf, dx_acc.at[pl.ds(s * RPT, RPT)])
    pltpu.sync_copy(v_hbm.at[pl.ds(s * RPT, RPT)], v_sh.at[pl.ds(s * RPT, RPT)])
    plsc.subcore_barrier()

    base = wid * 400
    nk = 10

    def chunk(k, _):
        off = base + k * 40
        pltpu.sync_copy(rxn2d.at[pl.ds(off, 40)], idx_r)
        pltpu.sync_copy(met2d.at[pl.ds(off, 40)], idx_m)
        pltpu.sync_copy(sto2d.at[pl.ds(off, 40)], sto_v)

        def grow(j, _):
            pltpu.async_copy(v_sh.at[idx_r.at[j]], val_v.at[j], sem)
            return 0
        lax.fori_loop(0, 40, grow, 0)
        pltpu.make_async_copy(sto2d.at[pl.ds(0, 40)], val_v, sem).wait()

        def fmul(i, _):
            r = i // 8
            l = (i % 8) * 16
            val_v[r, pl.ds(l, 16)] = val_v[r, pl.ds(l, 16)] * sto_v[r, pl.ds(l, 16)]
            return 0
        lax.fori_loop(0, 320, fmul, 0)

        def srow(j, _):
            pltpu.async_copy(val_v.at[j], dx_acc.at[idx_m.at[j]], sem, add=True)
            return 0
        lax.fori_loop(0, 40, srow, 0)
        pltpu.make_async_copy(sto2d.at[pl.ds(0, 40)], val_v, sem).wait()
        return 0
    lax.fori_loop(0, nk, chunk, 0)
    plsc.subcore_barrier()
    pltpu.sync_copy(dx_acc.at[pl.ds(s * RPT, RPT)],
                    dxp.at[pl.ds(c * NM_PAD + s * RPT, RPT)])


def _msg_mlp_block(c_ref, s_ref, wc_ref, ws_ref, b1c_ref, W2_ref, out_ref):
    cr = c_ref[...]
    sr = s_ref[...]
    ones = jnp.ones(cr.shape, _f32)
    z = (jax.lax.dot_general(wc_ref[...], cr, (((1,), (0,)), ((), ())),
                             preferred_element_type=_f32)
         + jax.lax.dot_general(ws_ref[...], sr, (((1,), (0,)), ((), ())),
                               preferred_element_type=_f32)
         + jax.lax.dot_general(b1c_ref[...], ones, (((1,), (0,)), ((), ())),
                               preferred_element_type=_f32))
    h = jnp.tanh(z)
    out_ref[...] = jax.lax.dot_general(W2_ref[...], h, (((1,), (0,)), ((), ())),
                                       preferred_element_type=_f32)


def _rate_block(h0_ref, h1_ref, ee_ref, nn_ref, lk_ref,
                b2c_ref, V1a_ref, V2_ref, c2_ref, v_ref):
    ee = ee_ref[...]
    nn = nn_ref[...]
    n = nn[0:1, :] + nn[1:2, :]
    h = (h0_ref[...] + h1_ref[...]
         + jax.lax.dot_general(b2c_ref[...], n, (((1,), (0,)), ((), ())),
                               preferred_element_type=_f32))
    ones = jnp.ones(n.shape, _f32)
    h_aug = jnp.concatenate([h, ones], axis=0)
    t = jnp.tanh(jax.lax.dot_general(V1a_ref[...], h_aug, (((1,), (0,)), ((), ())),
                                     preferred_element_type=_f32))
    r = jax.lax.dot_general(V2_ref[...], t, (((1,), (0,)), ((), ())),
                            preferred_element_type=_f32) + c2_ref[0, 0]
    base_v = jax.nn.softplus(r)
    ext_mean = (ee[0:1, :] + ee[1:2, :]) / jnp.maximum(n, 1.0)
    k = jnp.exp(lk_ref[...] * jnp.log(10.0).astype(_f32))
    v_ref[...] = k * ext_mean * base_v


def _add_block(p_ref, o_ref):
    p = p_ref[...]
    o_ref[...] = p[0:1, :] + p[1:2, :]


def kernel(x, met_sub, rxn_sub, sto_sub, met_all, rxn_all, sto_all,
           W1, b1, W2, b2, V1, c1, V2, c2, log_k):
    conc = jnp.concatenate([x[:, 3], jnp.zeros((NR_PAD - N_MET,), _f32)])
    ext = jnp.concatenate([x[:, 4], jnp.zeros((NR_PAD - N_MET,), _f32)])

    ps = ES_PAD - E_SUB
    met2d_s = jnp.concatenate([met_sub, jnp.zeros((ps,), jnp.int32)]).reshape(ES_ROWS, 128)
    rxn2d_s = jnp.concatenate([rxn_sub, jnp.full((ps,), N_RXN, jnp.int32)]).reshape(ES_ROWS, 128)
    sto_s = jnp.concatenate([sto_sub, jnp.zeros((ps,), _f32)])
    pa = EA_PAD - E_ALL
    rxn2d_a = jnp.concatenate([rxn_all, jnp.zeros((pa,), jnp.int32)]).reshape(EA_ROWS, 128)
    met2d_a = jnp.concatenate([met_all, jnp.full((pa,), N_MET, jnp.int32)]).reshape(EA_ROWS, 128)
    sto2d_a = jnp.concatenate([sto_all, jnp.zeros((pa,), _f32)]).reshape(EA_ROWS, 128)
    lk_pad = jnp.concatenate([log_k, jnp.zeros((NR_PAD - N_RXN,), _f32)])

    kernel_a = pl.kernel(
        _gather_scatter_sub,
        out_type=(jax.ShapeDtypeStruct((ES_ROWS, 128), _f32),
                  jax.ShapeDtypeStruct((NC * NR_PAD,), _f32),
                  jax.ShapeDtypeStruct((NC * NR_PAD,), _f32)),
        mesh=_MESH,
        scratch_types=[
            pltpu.VMEM((40, 128), jnp.int32),
            pltpu.VMEM((40, 128), jnp.int32),
            pltpu.VMEM((40, 128), _f32),
            pltpu.VMEM((40, 128), _f32),
            pltpu.VMEM((40, 128), _f32),
            pltpu.VMEM((RPT,), _f32),
            pltpu.VMEM_SHARED((NR_PAD,), _f32),
            pltpu.VMEM_SHARED((NR_PAD,), _f32),
            pltpu.VMEM_SHARED((NR_PAD,), _f32),
            pltpu.VMEM_SHARED((NR_PAD,), _f32),
            pltpu.SemaphoreType.DMA,
        ],
    )
    concg2d, extp, cntp = kernel_a(conc, ext, met2d_s, rxn2d_s)
    concg = concg2d.reshape(ES_PAD)

    BE = 16384
    msg_t = pl.pallas_call(
        _msg_mlp_block,
        grid=(ES_PAD // BE,),
        in_specs=[
            pl.BlockSpec((1, BE), lambda i: (0, i)),
            pl.BlockSpec((1, BE), lambda i: (0, i)),
            pl.BlockSpec((32, 1), lambda i: (0, 0)),
            pl.BlockSpec((32, 1), lambda i: (0, 0)),
            pl.BlockSpec((32, 1), lambda i: (0, 0)),
            pl.BlockSpec((16, 32), lambda i: (0, 0)),
        ],
        out_specs=pl.BlockSpec((16, BE), lambda i: (0, i)),
        out_shape=jax.ShapeDtypeStruct((16, ES_PAD), _f32),
    )(concg[None, :], sto_s[None, :], W1[:, 0:1], W1[:, 1:2], b1[:, None], W2)

    kernel_c = pl.kernel(
        _scatter_msg,
        out_type=jax.ShapeDtypeStruct((16, NC * NR_PAD), _f32),
        mesh=_MESH,
        scratch_types=(
            [pltpu.VMEM((8, 128), jnp.int32),
             pltpu.VMEM((16, 1024), _f32),
             pltpu.VMEM((RPT,), _f32)]
            + [pltpu.VMEM_SHARED((NR_PAD,), _f32) for _ in range(16)]
            + [pltpu.SemaphoreType.DMA]
        ),
    )
    hp = kernel_c(msg_t, rxn2d_s)

    V1aug = jnp.concatenate([V1, c1[:, None]], axis=1)
    extp2 = extp.reshape(NC, NR_PAD)
    cntp2 = cntp.reshape(NC, NR_PAD)
    BR = 2048
    v_t = pl.pallas_call(
        _rate_block,
        grid=(NR_PAD // BR,),
        in_specs=[
            pl.BlockSpec((16, BR), lambda i: (0, i)),
            pl.BlockSpec((16, BR), lambda i: (0, i)),
            pl.BlockSpec((2, BR), lambda i: (0, i)),
            pl.BlockSpec((2, BR), lambda i: (0, i)),
            pl.BlockSpec((1, BR), lambda i: (0, i)),
            pl.BlockSpec((16, 1), lambda i: (0, 0)),
            pl.BlockSpec((32, 17), lambda i: (0, 0)),
            pl.BlockSpec((1, 32), lambda i: (0, 0)),
            pl.BlockSpec((1, 1), lambda i: (0, 0)),
        ],
        out_specs=pl.BlockSpec((1, BR), lambda i: (0, i)),
        out_shape=jax.ShapeDtypeStruct((1, NR_PAD), _f32),
    )(hp[:, :NR_PAD], hp[:, NR_PAD:], extp2, cntp2,
      lk_pad[None, :], b2[:, None], V1aug, V2, c2[:, None])

    kernel_e = pl.kernel(
        _scatter_flux,
        out_type=jax.ShapeDtypeStruct((NC * NM_PAD,), _f32),
        mesh=_MESH,
        scratch_types=[
            pltpu.VMEM((40, 128), jnp.int32),
            pltpu.VMEM((40, 128), jnp.int32),
            pltpu.VMEM((40, 128), _f32),
            pltpu.VMEM((40, 128), _f32),
            pltpu.VMEM((RPT,), _f32),
            pltpu.VMEM_SHARED((NR_PAD,), _f32),
            pltpu.VMEM_SHARED((NM_PAD,), _f32),
            pltpu.SemaphoreType.DMA,
        ],
    )
    dxp = kernel_e(v_t.reshape(NR_PAD), rxn2d_a, met2d_a, sto2d_a)

    BN = 2048
    dx = pl.pallas_call(
        _add_block,
        grid=(NM_PAD // BN,),
        in_specs=[pl.BlockSpec((2, BN), lambda i: (0, i))],
        out_specs=pl.BlockSpec((1, BN), lambda i: (0, i)),
        out_shape=jax.ShapeDtypeStruct((1, NM_PAD), _f32),
    )(dxp.reshape(NC, NM_PAD))

    return dx.reshape(NM_PAD)[:N_MET, None]
